# baseline (speedup 1.0000x reference)
import functools

import jax
import jax.numpy as jnp
from jax import lax
from jax.experimental import pallas as pl
from jax.experimental.pallas import tpu as pltpu
from jax.experimental.pallas import tpu_sc as plsc

L = 16


@functools.lru_cache(maxsize=None)
def _build_sc_scatter(M, D, N):
    NW = 32
    MB = (M // 128) * 128
    CH = 896
    SP = 31360
    SL = MB - (NW - 1) * SP
    assert SP % CH == 0 and SP % 128 == 0 and 0 < SL <= SP
    NCH_F, REM_F = divmod(SP, CH)
    NCH_L, REM_L = divmod(SL, CH)
    assert REM_F == 0 and REM_L == 0
    assert SP < (1 << 15) and N <= (1 << 15)
    SCAN_STEPS = N // L

    mesh = plsc.VectorSubcoreMesh(core_axis_name="c", subcore_axis_name="s")

    @functools.partial(
        pl.kernel,
        out_type=jax.ShapeDtypeStruct((D, M), jnp.float32),
        mesh=mesh,
        scratch_types=[
            pltpu.VMEM((N,), jnp.int32),
            pltpu.VMEM((N,), jnp.int32),
            pltpu.VMEM((56, 128), jnp.float32),
            pltpu.VMEM((56,), jnp.int32),
            pltpu.VMEM((D, CH), jnp.float32),
            pltpu.VMEM((D, CH), jnp.float32),
            pltpu.SemaphoreType.DMA,
            pltpu.SemaphoreType.DMA,
            pltpu.SemaphoreType.DMA,
            pltpu.SemaphoreType.DMA,
            pltpu.SemaphoreType.DMA,
        ],
        compiler_params=pltpu.CompilerParams(needs_layout_passes=False),
    )
    def k(in_hbm, idx_hbm, val_hbm, out_hbm, idx_v, list_v, rows_big,
          pos_ref, buf0, buf1, gsem, i0, i1, o0, o1):
        iota = lax.iota(jnp.int32, L)
        wid = lax.axis_index("s") * 2 + lax.axis_index("c")
        lo = wid * SP
        last = wid == NW - 1
        S = jnp.where(last, SL, SP)

        bufs = (buf0, buf1)
        in_sems = (i0, i1)
        out_sems = (o0, o1)

        with jax.named_scope("idx_stage"):
            pltpu.sync_copy(idx_hbm, idx_v)

        def scan_body(k_, count):
            v = idx_v[pl.ds(k_ * L, L)]
            rel = v - lo
            m = (rel >= 0) & (rel < S)
            pos = k_ * L + iota
            packed = (rel << 15) | pos
            cs = plsc.cumsum(jnp.where(m, 1, 0))
            dest = count + cs - 1
            plsc.store_scatter(list_v, [dest], packed, mask=m)
            pop = plsc.all_reduce_population_count(m)
            return count + pop[0]

        with jax.named_scope("scan"):
            count = lax.fori_loop(0, SCAN_STEPS, scan_body, jnp.int32(0),
                                  unroll=8)
        ngroups = (count + L - 1) // L
        clist = idx_v

        def filter_chunk(c_rel_lo, span):
            def fbody(g, ccount):
                e = list_v[pl.ds(g * L, L)]
                tok = (g * L + iota) < count
                rel = e >> 15
                m = tok & (rel >= c_rel_lo) & (rel < c_rel_lo + span)
                cs = plsc.cumsum(jnp.where(m, 1, 0))
                dest = ccount + cs - 1
                plsc.store_scatter(clist, [dest], e, mask=m)
                pop = plsc.all_reduce_population_count(m)
                return ccount + pop[0]

            return lax.fori_loop(0, ngroups, fbody, jnp.int32(0))

        def _bcast(v, jv):
            return v.at[jv].get(mode="promise_in_bounds")

        B = 56

        def fill_fire(bb, ccount):
            for u in range(4):
                base = bb * B + u * L
                e = clist[pl.ds(base, L)]
                ok = ((base + iota) < ccount) & (u * L + iota < B)
                pr = jnp.where(ok, (e & 0x7FFF) >> 2, 0)
                if u < 3:
                    pos_ref[pl.ds(u * L, L)] = pr
                else:
                    plsc.store_scatter(pos_ref, [48 + (iota & 7)], pr,
                                       mask=iota < 8)
            pltpu.make_async_copy(val_hbm.at[pos_ref], rows_big,
                                  gsem).start()

        def poke_batch(bb, ccount, buf, c_rel_lo):
            pltpu.make_async_copy(val_hbm.at[pos_ref], rows_big, gsem).wait()
            for u in range(4):
                base = bb * B + u * L

                @pl.when(base < ccount)
                def _():
                    e = clist[pl.ds(base, L)]
                    tok = ((base + iota) < ccount) & (u * L + iota < B)
                    relv = e >> 15
                    posv = e & 0x7FFF
                    colv = jnp.clip(relv - c_rel_lo, 0, CH - 1)
                    tv = jnp.where(tok, 1, 0)
                    wp = posv
                    wl = iota
                    for j in range(L):
                        jv = jnp.full((L,), j, jnp.int32)
                        rj = _bcast(relv, jv)
                        pj = _bcast(posv, jv)
                        tj = _bcast(tv, jv)
                        upd = (relv == rj) & (tj > 0) & tok & (pj > wp)
                        wp = jnp.where(upd, pj, wp)
                        wl = jnp.where(upd, jv, wl)
                    qoff = (wp & 3) * 32
                    for d in range(D):
                        data = plsc.load_gather(rows_big,
                                                [u * L + wl, qoff + d],
                                                mask=tok)
                        plsc.store_scatter(buf,
                                           [jnp.full((L,), d, jnp.int32),
                                            colv], data, mask=tok)

        def apply_batches(ccount, buf, c_rel_lo):
            nbat = (ccount + B - 1) // B

            def bbody(bb, carry):
                poke_batch(bb, ccount, buf, c_rel_lo)

                @pl.when(bb + 1 < nbat)
                def _():
                    fill_fire(bb + 1, ccount)

                return carry

            lax.fori_loop(0, nbat, bbody, 0)

        def sweep(n):
            def in_cp(b, c):
                return pltpu.make_async_copy(
                    in_hbm.at[:, pl.ds(lo + c * CH, CH)], bufs[b],
                    in_sems[b])

            def out_cp(b, c):
                return pltpu.make_async_copy(
                    bufs[b], out_hbm.at[:, pl.ds(lo + c * CH, CH)],
                    out_sems[b])

            in_cp(0, 0).start()
            T = (n + 1) // 2

            def obody(t, carry):
                c0 = t * 2
                for b in range(2):
                    c = c0 + b

                    @pl.when(c < n)
                    def _():
                        ccount = filter_chunk(c * CH, CH)

                        @pl.when(ccount > 0)
                        def _():
                            fill_fire(jnp.int32(0), ccount)

                        in_cp(b, c).wait()

                        @pl.when(c + 1 < n)
                        def _():
                            @pl.when(c >= 1)
                            def _():
                                out_cp((b + 1) % 2, c - 1).wait()

                            in_cp((b + 1) % 2, c + 1).start()

                        @pl.when(ccount > 0)
                        def _():
                            apply_batches(ccount, bufs[b], c * CH)

                        out_cp(b, c).start()

                return carry

            lax.fori_loop(0, T, obody, 0)
            for c_last in (n - 2, n - 1):
                if c_last >= 0:
                    out_cp(c_last % 2, c_last).wait()

        with jax.named_scope("sweep"):
            @pl.when(jnp.logical_not(last))
            def _():
                sweep(NCH_F)

            @pl.when(last)
            def _():
                sweep(NCH_L)

    return k


def kernel(input, indices, values, accumulate):
    M, D = input.shape
    idx_flat = indices.reshape(-1)
    val_flat = values.reshape(-1, D)
    N = idx_flat.shape[0]
    assert D == 32 and N % 4 == 0

    k = _build_sc_scatter(M, D, N)
    val_rs = val_flat.reshape(N // 4, 4 * D)
    out_t = k(input.T, idx_flat, val_rs)
    out = out_t.T

    MB = (M // 128) * 128
    TAIL = M - MB
    if TAIL:
        pos = jnp.arange(N, dtype=jnp.int32)[:, None]
        match = idx_flat[:, None] == (MB + jnp.arange(TAIL, dtype=jnp.int32))
        lastpos = jnp.max(jnp.where(match, pos, -1), axis=0)
        has = lastpos >= 0
        tail_rows = jnp.where(has[:, None],
                              val_flat[jnp.clip(lastpos, 0, N - 1)],
                              input[MB:])
        out = lax.dynamic_update_slice(out, tail_rows, (MB, 0))
    return out


# speedup vs baseline: 2.4488x; 2.4488x over previous
import functools

import jax
import jax.numpy as jnp
from jax import lax
from jax.experimental import pallas as pl
from jax.experimental.pallas import tpu as pltpu
from jax.experimental.pallas import tpu_sc as plsc

L = 16


@functools.lru_cache(maxsize=None)
def _build_sc_scatter(M, D, N):
    NW = 32
    MB = (M // 128) * 128
    CH = 896
    SP = 31360
    SL = MB - (NW - 1) * SP
    assert SP % CH == 0 and SP % 128 == 0 and 0 < SL <= SP
    NCH_F, REM_F = divmod(SP, CH)
    NCH_L, REM_L = divmod(SL, CH)
    assert REM_F == 0 and REM_L == 0
    assert SP < (1 << 15) and N <= (1 << 15)
    SCAN_STEPS = N // L

    mesh = plsc.VectorSubcoreMesh(core_axis_name="c", subcore_axis_name="s")

    @functools.partial(
        pl.kernel,
        out_type=jax.ShapeDtypeStruct((D, M), jnp.float32),
        mesh=mesh,
        scratch_types=[
            pltpu.VMEM((N,), jnp.int32),
            pltpu.VMEM((N,), jnp.int32),
            pltpu.VMEM((L, 128), jnp.float32),
            pltpu.VMEM((D, CH), jnp.float32),
            pltpu.VMEM((D, CH), jnp.float32),
            pltpu.SemaphoreType.DMA,
            pltpu.SemaphoreType.DMA,
            pltpu.SemaphoreType.DMA,
            pltpu.SemaphoreType.DMA,
            pltpu.SemaphoreType.DMA,
        ],
        compiler_params=pltpu.CompilerParams(needs_layout_passes=False),
    )
    def k(in_hbm, idx_hbm, val_hbm, out_hbm, idx_v, list_v, rows_v,
          buf0, buf1, gsem, i0, i1, o0, o1):
        iota = lax.iota(jnp.int32, L)
        wid = lax.axis_index("s") * 2 + lax.axis_index("c")
        lo = wid * SP
        last = wid == NW - 1
        S = jnp.where(last, SL, SP)

        bufs = (buf0, buf1)
        in_sems = (i0, i1)
        out_sems = (o0, o1)

        with jax.named_scope("idx_stage"):
            pltpu.sync_copy(idx_hbm, idx_v)

        def scan_body(k_, count):
            v = idx_v[pl.ds(k_ * L, L)]
            rel = v - lo
            m = (rel >= 0) & (rel < S)
            pos = k_ * L + iota
            packed = (rel << 15) | pos
            cs = plsc.cumsum(jnp.where(m, 1, 0))
            dest = count + cs - 1
            plsc.store_scatter(list_v, [dest], packed, mask=m)
            pop = plsc.all_reduce_population_count(m)
            return count + pop[0]

        with jax.named_scope("scan"):
            count = lax.fori_loop(0, SCAN_STEPS, scan_body, jnp.int32(0),
                                  unroll=8)
        ngroups = (count + L - 1) // L
        clist = idx_v

        def filter_chunk(c_rel_lo, span):
            def fbody(g, ccount):
                e = list_v[pl.ds(g * L, L)]
                tok = (g * L + iota) < count
                rel = e >> 15
                m = tok & (rel >= c_rel_lo) & (rel < c_rel_lo + span)
                cs = plsc.cumsum(jnp.where(m, 1, 0))
                dest = ccount + cs - 1
                plsc.store_scatter(clist, [dest], e, mask=m)
                pop = plsc.all_reduce_population_count(m)
                return ccount + pop[0]

            return lax.fori_loop(0, ngroups, fbody, jnp.int32(0))

        def _bcast(v, jv):
            return v.at[jv].get(mode="promise_in_bounds")

        def apply_chunk(buf, c_rel_lo, ccount):
            cgroups = (ccount + L - 1) // L

            def abody(h, carry):
                e = clist[pl.ds(h * L, L)]
                tok = (h * L + iota) < ccount
                relv = e >> 15
                posv = e & 0x7FFF
                posc = jnp.where(tok, posv, 0)
                gcp = pltpu.make_async_copy(
                    val_hbm.at[posc >> 2], rows_v, gsem)
                gcp.start()
                gcp.wait()
                colv = jnp.clip(relv - c_rel_lo, 0, CH - 1)
                tv = jnp.where(tok, 1, 0)
                wp = posv
                wl = iota
                for j in range(L):
                    jv = jnp.full((L,), j, jnp.int32)
                    rj = _bcast(relv, jv)
                    pj = _bcast(posv, jv)
                    tj = _bcast(tv, jv)
                    upd = (relv == rj) & (tj > 0) & tok & (pj > wp)
                    wp = jnp.where(upd, pj, wp)
                    wl = jnp.where(upd, jv, wl)
                qoff = (wp & 3) * 32
                for d in range(D):
                    data = plsc.load_gather(rows_v, [wl, qoff + d],
                                            mask=tok)
                    plsc.store_scatter(buf,
                                       [jnp.full((L,), d, jnp.int32),
                                        colv], data, mask=tok)
                return carry

            lax.fori_loop(0, cgroups, abody, 0)

        def sweep(n):
            def in_cp(b, c):
                return pltpu.make_async_copy(
                    in_hbm.at[:, pl.ds(lo + c * CH, CH)], bufs[b],
                    in_sems[b])

            def out_cp(b, c):
                return pltpu.make_async_copy(
                    bufs[b], out_hbm.at[:, pl.ds(lo + c * CH, CH)],
                    out_sems[b])

            in_cp(0, 0).start()
            T = (n + 1) // 2

            def obody(t, carry):
                c0 = t * 2
                for b in range(2):
                    c = c0 + b

                    @pl.when(c < n)
                    def _():
                        in_cp(b, c).wait()

                        @pl.when(c + 1 < n)
                        def _():
                            @pl.when(c >= 1)
                            def _():
                                out_cp((b + 1) % 2, c - 1).wait()

                            in_cp((b + 1) % 2, c + 1).start()

                        ccount = filter_chunk(c * CH, CH)

                        @pl.when(ccount > 0)
                        def _():
                            apply_chunk(bufs[b], c * CH, ccount)

                        out_cp(b, c).start()

                return carry

            lax.fori_loop(0, T, obody, 0)
            for c_last in (n - 2, n - 1):
                if c_last >= 0:
                    out_cp(c_last % 2, c_last).wait()

        with jax.named_scope("sweep"):
            @pl.when(jnp.logical_not(last))
            def _():
                sweep(NCH_F)

            @pl.when(last)
            def _():
                sweep(NCH_L)

    return k


def kernel(input, indices, values, accumulate):
    M, D = input.shape
    idx_flat = indices.reshape(-1)
    val_flat = values.reshape(-1, D)
    N = idx_flat.shape[0]
    assert D == 32 and N % 4 == 0

    k = _build_sc_scatter(M, D, N)
    val_rs = val_flat.reshape(N // 4, 4 * D)
    out_t = k(input.T, idx_flat, val_rs)
    out = out_t.T

    MB = (M // 128) * 128
    TAIL = M - MB
    if TAIL:
        pos = jnp.arange(N, dtype=jnp.int32)[:, None]
        match = idx_flat[:, None] == (MB + jnp.arange(TAIL, dtype=jnp.int32))
        lastpos = jnp.max(jnp.where(match, pos, -1), axis=0)
        has = lastpos >= 0
        tail_rows = jnp.where(has[:, None],
                              val_flat[jnp.clip(lastpos, 0, N - 1)],
                              input[MB:])
        out = lax.dynamic_update_slice(out, tail_rows, (MB, 0))
    return out
